# Initial kernel scaffold; baseline (speedup 1.0000x reference)
import jax
import jax.numpy as jnp
from jax.experimental import pallas as pl


def kernel(timestamps, table_0, table_1, table_2, table_3, table_4, table_5, table_6):
    raise NotImplementedError("write your pallas kernel here")


# speedup vs baseline: 2.9035x; 2.9035x over previous
import functools

import jax
import jax.numpy as jnp
from jax import lax
from jax.experimental import pallas as pl
from jax.experimental.pallas import tpu as pltpu
from jax.experimental.pallas import tpu_sc as plsc

_SIZES = (60, 60, 24, 7, 31, 12, 366)
_HIDDEN = 128
_NC, _NS = 2, 16
_NW = _NC * _NS


@functools.lru_cache(maxsize=None)
def _make_sc_kernel(n_total: int):
    n_per_w = n_total // _NW
    rows = 128
    n_chunks = n_per_w // rows
    mesh = plsc.VectorSubcoreMesh(
        core_axis_name="c", subcore_axis_name="s",
        num_cores=_NC, num_subcores=_NS,
    )

    @functools.partial(
        pl.kernel,
        out_type=jax.ShapeDtypeStruct((n_total, _HIDDEN), jnp.float32),
        mesh=mesh,
        scratch_types=[
            pltpu.VMEM((7, n_per_w), jnp.float32),
            pltpu.VMEM((n_chunks, 7, rows), jnp.int32),
            pltpu.VMEM((rows, _HIDDEN), jnp.float32),
            pltpu.SemaphoreType.DMA,
        ],
    )
    def sc_kernel(ts_hbm, t0, t1, t2, t3, t4, t5, t6, out_hbm,
                  ts_v, idx_v, acc_v, sem):
        tables = (t0, t1, t2, t3, t4, t5, t6)
        wid = lax.axis_index("s") * _NC + lax.axis_index("c")
        base = wid * n_per_w
        pltpu.sync_copy(ts_hbm.at[:, pl.ds(base, n_per_w)], ts_v)

        def idx_body(c, carry):
            for i in range(7):
                for k in range(rows // 16):
                    v = ts_v[i, pl.ds(c * rows + k * 16, 16)]
                    idx_v[c, i, pl.ds(k * 16, 16)] = (
                        v * jnp.float32(_SIZES[i])).astype(jnp.int32)
            return carry
        lax.fori_loop(0, n_chunks, idx_body, 0)

        def gather_body(c, carry):
            pltpu.async_copy(tables[0].at[idx_v.at[c, 0]], acc_v, sem).wait()
            for i in range(1, 7):
                pltpu.async_copy(
                    tables[i].at[idx_v.at[c, i]], acc_v, sem, add=True).wait()
            pltpu.sync_copy(acc_v, out_hbm.at[pl.ds(base + c * rows, rows)])
            return carry
        lax.fori_loop(0, n_chunks, gather_body, 0)

    return sc_kernel


def kernel(timestamps, table_0, table_1, table_2, table_3, table_4,
           table_5, table_6):
    b, s, f = timestamps.shape
    n_total = b * s
    ts_t = timestamps.reshape(n_total, f).T
    out = _make_sc_kernel(n_total)(
        ts_t, table_0, table_1, table_2, table_3, table_4, table_5, table_6)
    return out.reshape(b, s, _HIDDEN)


# speedup vs baseline: 21.9450x; 7.5580x over previous
import functools

import jax
import jax.numpy as jnp
from jax import lax
from jax.experimental import pallas as pl
from jax.experimental.pallas import tpu as pltpu
from jax.experimental.pallas import tpu_sc as plsc

_SIZES = (60, 60, 24, 7, 31, 12, 366)
_HIDDEN = 128
_NC, _NS = 2, 16
_NW = _NC * _NS


def _build_product_tables(t0, t1, t2, t3, t4, t5, t6):
    f32 = jnp.float32

    def body(t0r, t1r, t2r, t3r, t4r, t5r, t6r, oa, ob, oc):
        oa[...] = t0r[:][:, None, :] + t1r[:][None, :, :]
        ob[...] = (t2r[:][:, None, None, :] + t3r[:][None, :, None, :]
                   ) + t5r[:][None, None, :, :]
        oc[...] = t4r[:][:, None, :] + t6r[:][None, :, :]

    oa, ob, oc = pl.pallas_call(
        body,
        out_shape=[
            jax.ShapeDtypeStruct((60, 60, _HIDDEN), f32),
            jax.ShapeDtypeStruct((24, 7, 12, _HIDDEN), f32),
            jax.ShapeDtypeStruct((31, 366, _HIDDEN), f32),
        ],
    )(t0, t1, t2, t3, t4, t5, t6)
    return (oa.reshape(3600, _HIDDEN), ob.reshape(2016, _HIDDEN),
            oc.reshape(11346, _HIDDEN))


@functools.lru_cache(maxsize=None)
def _make_sc_kernel(n_total: int):
    n_per_w = n_total // _NW
    rows = 256
    n_chunks = n_per_w // rows
    mesh = plsc.VectorSubcoreMesh(
        core_axis_name="c", subcore_axis_name="s",
        num_cores=_NC, num_subcores=_NS,
    )

    @functools.partial(
        pl.kernel,
        out_type=jax.ShapeDtypeStruct((n_total, _HIDDEN), jnp.float32),
        mesh=mesh,
        scratch_types=[
            pltpu.VMEM((7, rows), jnp.float32),
            pltpu.VMEM((7, rows), jnp.float32),
            pltpu.VMEM((n_chunks, 3, 2, 128), jnp.int32),
            pltpu.VMEM((rows, _HIDDEN), jnp.float32),
            pltpu.VMEM((rows, _HIDDEN), jnp.float32),
            pltpu.VMEM((rows, _HIDDEN), jnp.float32),
            pltpu.SemaphoreType.DMA,
            pltpu.SemaphoreType.DMA,
            pltpu.SemaphoreType.DMA,
            pltpu.SemaphoreType.DMA,
            pltpu.SemaphoreType.DMA,
            pltpu.SemaphoreType.DMA,
            pltpu.SemaphoreType.DMA,
            pltpu.SemaphoreType.DMA,
            pltpu.SemaphoreType.DMA,
            pltpu.SemaphoreType.DMA,
            pltpu.SemaphoreType.DMA,
        ],
    )
    def sc_kernel(ts_hbm, ta, tb, tc, out_hbm,
                  tsb0, tsb1, idx_v, acc0, acc1, acc2,
                  ts0, ts1, g0s0, g0s1, g0s2, as0, as1, as2, ws0, ws1, ws2):
        tsb = (tsb0, tsb1)
        tsem = (ts0, ts1)
        accs = (acc0, acc1, acc2)
        g0sem = (g0s0, g0s1, g0s2)
        asem = (as0, as1, as2)
        wsem = (ws0, ws1, ws2)
        wid = lax.axis_index("s") * _NC + lax.axis_index("c")
        base = wid * n_per_w

        def ts_load(c, p):
            return pltpu.async_copy(
                ts_hbm.at[:, pl.ds(base + c * rows, rows)], tsb[p], tsem[p])

        tdesc = [ts_load(0, 0), None]
        for c in range(n_chunks):
            p = c & 1
            if c + 1 < n_chunks:
                tdesc[1 - p] = ts_load(c + 1, 1 - p)
            tdesc[p].wait()
            src = tsb[p]

            def idx_grp(k, carry):
                o = k * 16

                def tix(i):
                    return (src[i, pl.ds(o, 16)]
                            * jnp.float32(_SIZES[i])).astype(jnp.int32)

                ia = tix(0) * 60 + tix(1)
                ib = tix(2) * 84 + tix(3) * 12 + tix(5)
                ic = tix(4) * 366 + tix(6)
                j = k // 8
                sl = pl.ds((k % 8) * 16, 16)
                idx_v[c, 0, j, sl] = ia
                idx_v[c, 1, j, sl] = ib
                idx_v[c, 2, j, sl] = ic
                return carry
            lax.fori_loop(0, rows // 16, idx_grp, 0)

        def issue_g0(c):
            p = c % 3
            return [pltpu.async_copy(
                ta.at[idx_v.at[c, 0, j]],
                accs[p].at[pl.ds(j * 128, 128)], g0sem[p])
                for j in range(2)]

        def issue_adds(c):
            p = c % 3
            return [pltpu.async_copy(
                tab.at[idx_v.at[c, i, j]],
                accs[p].at[pl.ds(j * 128, 128)], asem[p], add=True)
                for i, tab in ((1, tb), (2, tc)) for j in range(2)]

        def issue_write(c):
            p = c % 3
            return pltpu.async_copy(
                accs[p], out_hbm.at[pl.ds(base + c * rows, rows)], wsem[p])

        gdesc = [None, None, None]
        adesc = [None, None, None]
        wdesc = [None, None, None]
        for c in range(n_chunks + 2):
            if c < n_chunks:
                p = c % 3
                if wdesc[p] is not None:
                    wdesc[p].wait()
                    wdesc[p] = None
                gdesc[p] = issue_g0(c)
            if 1 <= c <= n_chunks:
                q = (c - 1) % 3
                for d in gdesc[q]:
                    d.wait()
                adesc[q] = issue_adds(c - 1)
            if c >= 2:
                r = (c - 2) % 3
                for d in adesc[r]:
                    d.wait()
                wdesc[r] = issue_write(c - 2)
        for d in wdesc:
            if d is not None:
                d.wait()

    return sc_kernel


def kernel(timestamps, table_0, table_1, table_2, table_3, table_4,
           table_5, table_6):
    b, s, f = timestamps.shape
    n_total = b * s
    ta, tb, tc = _build_product_tables(
        table_0, table_1, table_2, table_3, table_4, table_5, table_6)
    ts_t = timestamps.reshape(n_total, f).T
    out = _make_sc_kernel(n_total)(ts_t, ta, tb, tc)
    return out.reshape(b, s, _HIDDEN)


# speedup vs baseline: 23.4623x; 1.0691x over previous
import functools

import jax
import jax.numpy as jnp
from jax import lax
from jax.experimental import pallas as pl
from jax.experimental.pallas import tpu as pltpu
from jax.experimental.pallas import tpu_sc as plsc

_SIZES = (60, 60, 24, 7, 31, 12, 366)
_HIDDEN = 128
_NC, _NS = 2, 16
_NW = _NC * _NS
_ROWS_A, _ROWS_B, _ROWS_C = 3600, 24 * 7 * 31, 12 * 366
_ROWS_ALL = _ROWS_A + _ROWS_B + _ROWS_C
_DEPTH = 6


def _build_product_tables(t0, t1, t2, t3, t4, t5, t6):
    def body(t0r, t1r, t2r, t3r, t4r, t5r, t6r, out):
        a = t0r[:][:, None, :] + t1r[:][None, :, :]
        b = (t2r[:][:, None, None, :] + t3r[:][None, :, None, :]
             ) + t4r[:][None, None, :, :]
        c = t5r[:][:, None, :] + t6r[:][None, :, :]
        out[pl.ds(0, _ROWS_A)] = a.reshape(_ROWS_A, _HIDDEN)
        out[pl.ds(_ROWS_A, _ROWS_B)] = b.reshape(_ROWS_B, _HIDDEN)
        out[pl.ds(_ROWS_A + _ROWS_B, _ROWS_C)] = c.reshape(_ROWS_C, _HIDDEN)

    return pl.pallas_call(
        body,
        out_shape=jax.ShapeDtypeStruct((_ROWS_ALL, _HIDDEN), jnp.float32),
    )(t0, t1, t2, t3, t4, t5, t6)


@functools.lru_cache(maxsize=None)
def _make_sc_kernel(n_total: int):
    n_per_w = n_total // _NW
    rows = 128
    n_chunks = n_per_w // rows
    mesh = plsc.VectorSubcoreMesh(
        core_axis_name="c", subcore_axis_name="s",
        num_cores=_NC, num_subcores=_NS,
    )

    @functools.partial(
        pl.kernel,
        out_type=jax.ShapeDtypeStruct((n_total, _HIDDEN), jnp.float32),
        mesh=mesh,
        scratch_types=(
            [pltpu.VMEM((7, rows), jnp.float32)] * 2
            + [pltpu.VMEM((n_chunks, 3, 128), jnp.int32)]
            + [pltpu.VMEM((rows, _HIDDEN), jnp.float32)] * _DEPTH
            + [pltpu.SemaphoreType.DMA] * 2
            + [pltpu.SemaphoreType.DMA] * _DEPTH
            + [pltpu.SemaphoreType.DMA] * _DEPTH
            + [pltpu.SemaphoreType.DMA] * _DEPTH
        ),
    )
    def sc_kernel(ts_hbm, tall, out_hbm, tsb0, tsb1, idx_v, *rest):
        accs = rest[:_DEPTH]
        g0sem = rest[_DEPTH + 2:2 * _DEPTH + 2]
        asem = rest[2 * _DEPTH + 2:3 * _DEPTH + 2]
        wsem = rest[3 * _DEPTH + 2:4 * _DEPTH + 2]
        tsb = (tsb0, tsb1)
        tsem = rest[_DEPTH:_DEPTH + 2]
        wid = lax.axis_index("s") * _NC + lax.axis_index("c")
        base = wid * n_per_w

        def ts_load(c, p):
            return pltpu.async_copy(
                ts_hbm.at[:, pl.ds(base + c * rows, rows)], tsb[p], tsem[p])

        tdesc = [ts_load(0, 0), None]
        for c in range(n_chunks):
            p = c & 1
            if c + 1 < n_chunks:
                tdesc[1 - p] = ts_load(c + 1, 1 - p)
            tdesc[p].wait()
            src = tsb[p]

            def idx_grp(k, carry):
                def tix(i):
                    v = src[i, pl.ds(k * 16, 16)]
                    return (v * jnp.float32(_SIZES[i])).astype(jnp.int32)

                ia = tix(0) * 60 + tix(1)
                ib = (tix(2) * 217 + tix(3) * 31 + tix(4)) + _ROWS_A
                ic = (tix(5) * 366 + tix(6)) + (_ROWS_A + _ROWS_B)
                sl = pl.ds(k * 16, 16)
                idx_v[c, 0, sl] = ia
                idx_v[c, 1, sl] = ib
                idx_v[c, 2, sl] = ic
                return carry
            lax.fori_loop(0, rows // 16, idx_grp, 0)

        def issue_g0(c):
            p = c % _DEPTH
            return pltpu.async_copy(
                tall.at[idx_v.at[c, 0]], accs[p], g0sem[p])

        def issue_adds(c):
            p = c % _DEPTH
            return [pltpu.async_copy(
                tall.at[idx_v.at[c, i]], accs[p], asem[p], add=True)
                for i in (1, 2)]

        def issue_write(c):
            p = c % _DEPTH
            return pltpu.async_copy(
                accs[p], out_hbm.at[pl.ds(base + c * rows, rows)], wsem[p])

        gdesc = [None] * _DEPTH
        adesc = [None] * _DEPTH
        wdesc = [None] * _DEPTH
        for c in range(n_chunks + 2):
            if c < n_chunks:
                p = c % _DEPTH
                if wdesc[p] is not None:
                    wdesc[p].wait()
                    wdesc[p] = None
                gdesc[p] = issue_g0(c)
            if 1 <= c <= n_chunks:
                q = (c - 1) % _DEPTH
                gdesc[q].wait()
                adesc[q] = issue_adds(c - 1)
            if c >= 2:
                r = (c - 2) % _DEPTH
                for d in adesc[r]:
                    d.wait()
                wdesc[r] = issue_write(c - 2)
        for d in wdesc:
            if d is not None:
                d.wait()

    return sc_kernel


def kernel(timestamps, table_0, table_1, table_2, table_3, table_4,
           table_5, table_6):
    b, s, f = timestamps.shape
    n_total = b * s
    tall = _build_product_tables(
        table_0, table_1, table_2, table_3, table_4, table_5, table_6)
    ts_t = timestamps.reshape(n_total, f).T
    out = _make_sc_kernel(n_total)(ts_t, tall)
    return out.reshape(b, s, _HIDDEN)


# speedup vs baseline: 25.0387x; 1.0672x over previous
import functools

import jax
import jax.numpy as jnp
from jax import lax
from jax.experimental import pallas as pl
from jax.experimental.pallas import tpu as pltpu
from jax.experimental.pallas import tpu_sc as plsc

_SIZES = (60, 60, 24, 7, 31, 12, 366)
_HIDDEN = 128
_NC, _NS = 2, 16
_NW = _NC * _NS
_ROWS_A, _ROWS_B, _ROWS_C = 3600, 24 * 7 * 31, 12 * 366
_ROWS_ALL = _ROWS_A + _ROWS_B + _ROWS_C
_DEPTH = 6


def _build_product_tables(t0, t1, t2, t3, t4, t5, t6):
    def body(t0r, t1r, t2r, t3r, t4r, t5r, t6r, out):
        a = t0r[:][:, None, :] + t1r[:][None, :, :]
        b = (t2r[:][:, None, None, :] + t3r[:][None, :, None, :]
             ) + t4r[:][None, None, :, :]
        c = t5r[:][:, None, :] + t6r[:][None, :, :]
        out[pl.ds(0, _ROWS_A)] = a.reshape(_ROWS_A, _HIDDEN)
        out[pl.ds(_ROWS_A, _ROWS_B)] = b.reshape(_ROWS_B, _HIDDEN)
        out[pl.ds(_ROWS_A + _ROWS_B, _ROWS_C)] = c.reshape(_ROWS_C, _HIDDEN)

    return pl.pallas_call(
        body,
        out_shape=jax.ShapeDtypeStruct((_ROWS_ALL, _HIDDEN), jnp.float32),
    )(t0, t1, t2, t3, t4, t5, t6)


@functools.lru_cache(maxsize=None)
def _make_sc_kernel(n_total: int):
    n_per_w = n_total // _NW
    rows = 128
    n_chunks = n_per_w // rows
    mesh = plsc.VectorSubcoreMesh(
        core_axis_name="c", subcore_axis_name="s",
        num_cores=_NC, num_subcores=_NS,
    )

    ring = 4

    @functools.partial(
        pl.kernel,
        out_type=jax.ShapeDtypeStruct((n_total, _HIDDEN), jnp.float32),
        mesh=mesh,
        scratch_types=(
            [pltpu.VMEM((7, rows), jnp.float32)] * ring
            + [pltpu.VMEM((n_chunks, 3, 128), jnp.int32)]
            + [pltpu.VMEM((rows, _HIDDEN), jnp.float32)] * _DEPTH
            + [pltpu.SemaphoreType.DMA] * ring
            + [pltpu.SemaphoreType.DMA] * _DEPTH
            + [pltpu.SemaphoreType.DMA] * _DEPTH
            + [pltpu.SemaphoreType.DMA] * _DEPTH
        ),
    )
    def sc_kernel(ts_hbm, tall, out_hbm, *rest):
        tsb = rest[:ring]
        idx_v = rest[ring]
        accs = rest[ring + 1:ring + 1 + _DEPTH]
        tsem = rest[ring + 1 + _DEPTH:2 * ring + 1 + _DEPTH]
        g0sem = rest[2 * ring + 1 + _DEPTH:2 * ring + 1 + 2 * _DEPTH]
        asem = rest[2 * ring + 1 + 2 * _DEPTH:2 * ring + 1 + 3 * _DEPTH]
        wsem = rest[2 * ring + 1 + 3 * _DEPTH:2 * ring + 1 + 4 * _DEPTH]
        wid = lax.axis_index("s") * _NC + lax.axis_index("c")
        base = wid * n_per_w

        def ts_load(c):
            p = c % ring
            return pltpu.async_copy(
                ts_hbm.at[:, pl.ds(base + c * rows, rows)], tsb[p], tsem[p])

        def compute_idx(c):
            src = tsb[c % ring]

            def idx_grp(k, carry):
                def tix(i):
                    v = src[i, pl.ds(k * 16, 16)]
                    return (v * jnp.float32(_SIZES[i])).astype(jnp.int32)

                ia = tix(0) * 60 + tix(1)
                ib = (tix(2) * 217 + tix(3) * 31 + tix(4)) + _ROWS_A
                ic = (tix(5) * 366 + tix(6)) + (_ROWS_A + _ROWS_B)
                sl = pl.ds(k * 16, 16)
                idx_v[c, 0, sl] = ia
                idx_v[c, 1, sl] = ib
                idx_v[c, 2, sl] = ic
                return carry
            lax.fori_loop(0, rows // 16, idx_grp, 0)

        def issue_g0(c):
            p = c % _DEPTH
            return pltpu.async_copy(
                tall.at[idx_v.at[c, 0]], accs[p], g0sem[p])

        def issue_adds(c):
            p = c % _DEPTH
            return [pltpu.async_copy(
                tall.at[idx_v.at[c, i]], accs[p], asem[p], add=True)
                for i in (1, 2)]

        def issue_write(c):
            p = c % _DEPTH
            return pltpu.async_copy(
                accs[p], out_hbm.at[pl.ds(base + c * rows, rows)], wsem[p])

        tdesc = [None] * ring
        for j in range(min(ring, n_chunks)):
            tdesc[j] = ts_load(j)
        for j in range(min(2, n_chunks)):
            tdesc[j].wait()
            compute_idx(j)

        gdesc = [None] * _DEPTH
        adesc = [None] * _DEPTH
        wdesc = [None] * _DEPTH
        for c in range(n_chunks + 2):
            if c + 2 < n_chunks:
                tdesc[(c + 2) % ring].wait()
                compute_idx(c + 2)
            if c + ring < n_chunks:
                tdesc[(c + ring) % ring] = ts_load(c + ring)
            if c < n_chunks:
                p = c % _DEPTH
                if wdesc[p] is not None:
                    wdesc[p].wait()
                    wdesc[p] = None
                gdesc[p] = issue_g0(c)
            if 1 <= c <= n_chunks:
                q = (c - 1) % _DEPTH
                gdesc[q].wait()
                adesc[q] = issue_adds(c - 1)
            if c >= 2:
                r = (c - 2) % _DEPTH
                for d in adesc[r]:
                    d.wait()
                wdesc[r] = issue_write(c - 2)
        for d in wdesc:
            if d is not None:
                d.wait()

    return sc_kernel


def kernel(timestamps, table_0, table_1, table_2, table_3, table_4,
           table_5, table_6):
    b, s, f = timestamps.shape
    n_total = b * s
    tall = _build_product_tables(
        table_0, table_1, table_2, table_3, table_4, table_5, table_6)
    ts_t = timestamps.reshape(n_total, f).T
    out = _make_sc_kernel(n_total)(ts_t, tall)
    return out.reshape(b, s, _HIDDEN)


# speedup vs baseline: 25.2483x; 1.0084x over previous
import functools

import jax
import jax.numpy as jnp
from jax import lax
from jax.experimental import pallas as pl
from jax.experimental.pallas import tpu as pltpu
from jax.experimental.pallas import tpu_sc as plsc

_SIZES = (60, 60, 24, 7, 31, 12, 366)
_HIDDEN = 128
_NC, _NS = 2, 16
_NW = _NC * _NS
_ROWS_A, _ROWS_B, _ROWS_C = 3600, 24 * 7 * 31, 12 * 366
_ROWS_ALL = _ROWS_A + _ROWS_B + _ROWS_C
_DEPTH = 6


def _build_product_tables(t0, t1, t2, t3, t4, t5, t6):
    def body(t0r, t1r, t2r, t3r, t4r, t5r, t6r, out):
        a = t0r[:][:, None, :] + t1r[:][None, :, :]
        b = (t2r[:][:, None, None, :] + t3r[:][None, :, None, :]
             ) + t4r[:][None, None, :, :]
        c = t5r[:][:, None, :] + t6r[:][None, :, :]
        out[pl.ds(0, _ROWS_A)] = a.reshape(_ROWS_A, _HIDDEN)
        out[pl.ds(_ROWS_A, _ROWS_B)] = b.reshape(_ROWS_B, _HIDDEN)
        out[pl.ds(_ROWS_A + _ROWS_B, _ROWS_C)] = c.reshape(_ROWS_C, _HIDDEN)

    return pl.pallas_call(
        body,
        out_shape=jax.ShapeDtypeStruct((_ROWS_ALL, _HIDDEN), jnp.float32),
    )(t0, t1, t2, t3, t4, t5, t6)


@functools.lru_cache(maxsize=None)
def _make_sc_kernel(n_total: int):
    n_per_w = n_total // _NW
    rows = 128
    n_chunks = n_per_w // rows
    mesh = plsc.VectorSubcoreMesh(
        core_axis_name="c", subcore_axis_name="s",
        num_cores=_NC, num_subcores=_NS,
    )

    ring = 4

    @functools.partial(
        pl.kernel,
        out_type=jax.ShapeDtypeStruct((n_total, _HIDDEN), jnp.float32),
        mesh=mesh,
        scratch_types=(
            [pltpu.VMEM((7, rows), jnp.float32)] * ring
            + [pltpu.VMEM((n_chunks, 3, 128), jnp.int32)]
            + [pltpu.VMEM((rows, _HIDDEN), jnp.float32)] * _DEPTH
            + [pltpu.SemaphoreType.DMA] * ring
            + [pltpu.SemaphoreType.DMA] * _DEPTH
            + [pltpu.SemaphoreType.DMA] * _DEPTH
            + [pltpu.SemaphoreType.DMA] * _DEPTH
        ),
    )
    def sc_kernel(ts_hbm, tall, out_hbm, *rest):
        tsb = rest[:ring]
        idx_v = rest[ring]
        accs = rest[ring + 1:ring + 1 + _DEPTH]
        tsem = rest[ring + 1 + _DEPTH:2 * ring + 1 + _DEPTH]
        g0sem = rest[2 * ring + 1 + _DEPTH:2 * ring + 1 + 2 * _DEPTH]
        asem = rest[2 * ring + 1 + 2 * _DEPTH:2 * ring + 1 + 3 * _DEPTH]
        wsem = rest[2 * ring + 1 + 3 * _DEPTH:2 * ring + 1 + 4 * _DEPTH]
        wid = lax.axis_index("s") * _NC + lax.axis_index("c")
        base = wid * n_per_w

        def ts_load(c):
            p = c % ring
            return pltpu.async_copy(
                ts_hbm.at[:, pl.ds(base + c * rows, rows)], tsb[p], tsem[p])

        def compute_idx(c):
            src = tsb[c % ring]

            def idx_grp(k, carry):
                def tix(i):
                    v = src[i, pl.ds(k * 16, 16)]
                    return (v * jnp.float32(_SIZES[i])).astype(jnp.int32)

                ia = tix(0) * 60 + tix(1)
                ib = (tix(2) * 217 + tix(3) * 31 + tix(4)) + _ROWS_A
                ic = (tix(5) * 366 + tix(6)) + (_ROWS_A + _ROWS_B)
                sl = pl.ds(k * 16, 16)
                idx_v[c, 0, sl] = ia
                idx_v[c, 1, sl] = ib
                idx_v[c, 2, sl] = ic
                return carry
            lax.fori_loop(0, rows // 16, idx_grp, 0)

        def issue_g0(c):
            p = c % _DEPTH
            return pltpu.async_copy(
                tall.at[idx_v.at[c, 0]], accs[p], g0sem[p])

        def issue_adds(c):
            p = c % _DEPTH
            return [pltpu.async_copy(
                tall.at[idx_v.at[c, i]], accs[p], asem[p], add=True)
                for i in (1, 2)]

        def issue_write(c):
            p = c % _DEPTH
            return pltpu.async_copy(
                accs[p], out_hbm.at[pl.ds(base + c * rows, rows)], wsem[p])

        tdesc = [None] * ring
        for j in range(min(ring, n_chunks)):
            tdesc[j] = ts_load(j)
        for j in range(min(2, n_chunks)):
            tdesc[j].wait()
            compute_idx(j)

        gdesc = [None] * _DEPTH
        adesc = [None] * _DEPTH
        wdesc = [None] * _DEPTH
        for c in range(n_chunks + 4):
            if c < n_chunks:
                p = c % _DEPTH
                if wdesc[p] is not None:
                    wdesc[p].wait()
                    wdesc[p] = None
                gdesc[p] = issue_g0(c)
            if 2 <= c < n_chunks + 2:
                q = (c - 2) % _DEPTH
                gdesc[q].wait()
                adesc[q] = issue_adds(c - 2)
            if c >= 4:
                r = (c - 4) % _DEPTH
                for d in adesc[r]:
                    d.wait()
                wdesc[r] = issue_write(c - 4)
            if c + 2 < n_chunks:
                tdesc[(c + 2) % ring].wait()
                compute_idx(c + 2)
            if c + ring < n_chunks:
                tdesc[(c + ring) % ring] = ts_load(c + ring)
        for d in wdesc:
            if d is not None:
                d.wait()

    return sc_kernel


def kernel(timestamps, table_0, table_1, table_2, table_3, table_4,
           table_5, table_6):
    b, s, f = timestamps.shape
    n_total = b * s
    tall = _build_product_tables(
        table_0, table_1, table_2, table_3, table_4, table_5, table_6)
    ts_t = timestamps.reshape(n_total, f).T
    out = _make_sc_kernel(n_total)(ts_t, tall)
    return out.reshape(b, s, _HIDDEN)
